# Initial kernel scaffold; baseline (speedup 1.0000x reference)
import functools

import jax
import jax.numpy as jnp
from jax import lax
from jax.experimental import pallas as pl
from jax.experimental.pallas import tpu as pltpu
from jax.experimental.pallas import tpu_sc as plsc

N_ROWS = 131072
N_FEAT = 128
NUM_CORES = 2
NUM_SUBCORES = 16
NUM_WORKERS = NUM_CORES * NUM_SUBCORES
ROWS_PER_WORKER = N_ROWS // NUM_WORKERS
CHUNK_ROWS = 512
NUM_CHUNKS = ROWS_PER_WORKER // CHUNK_ROWS
LANES = 16
BLOCKS = N_FEAT // LANES


def _permute_body(inputs_hbm, perm_hbm, out_hbm, perm_v, buf):
    c = lax.axis_index("c")
    s = lax.axis_index("s")
    wid = s * NUM_CORES + c
    base = wid * ROWS_PER_WORKER

    pltpu.sync_copy(perm_hbm, perm_v)
    idxs = [perm_v[pl.ds(k * LANES, LANES)] for k in range(BLOCKS)]

    def chunk_body(g, carry):
        row0 = base + g * CHUNK_ROWS
        pltpu.sync_copy(inputs_hbm.at[pl.ds(row0, CHUNK_ROWS)], buf)

        def row_body(r, carry2):
            row_vec = jnp.full((LANES,), r, jnp.int32)
            vals = [plsc.load_gather(buf, [row_vec, idxs[k]])
                    for k in range(BLOCKS)]
            for k in range(BLOCKS):
                buf[r, pl.ds(k * LANES, LANES)] = vals[k]
            return carry2

        lax.fori_loop(0, CHUNK_ROWS, row_body, 0)
        pltpu.sync_copy(buf, out_hbm.at[pl.ds(row0, CHUNK_ROWS)])
        return carry

    lax.fori_loop(0, NUM_CHUNKS, chunk_body, 0)


@jax.jit
def _permute(inputs, permutation):
    mesh = plsc.VectorSubcoreMesh(core_axis_name="c", subcore_axis_name="s")
    fn = functools.partial(
        pl.kernel,
        out_type=jax.ShapeDtypeStruct((N_ROWS, N_FEAT), jnp.float32),
        mesh=mesh,
        scratch_types=[
            pltpu.VMEM((N_FEAT,), jnp.int32),
            pltpu.VMEM((CHUNK_ROWS, N_FEAT), jnp.float32),
        ],
    )(_permute_body)
    return fn(inputs, permutation)


def kernel(inputs, permutation):
    outputs = _permute(inputs, permutation)
    logabsdet = jnp.zeros((inputs.shape[0],), dtype=inputs.dtype)
    return (outputs, logabsdet)


# speedup vs baseline: 2.5715x; 2.5715x over previous
import functools

import jax
import jax.numpy as jnp
from jax import lax
from jax.experimental import pallas as pl
from jax.experimental.pallas import tpu as pltpu
from jax.experimental.pallas import tpu_sc as plsc

N_ROWS = 131072
N_FEAT = 128
NUM_CORES = 2
NUM_SUBCORES = 16
NUM_WORKERS = NUM_CORES * NUM_SUBCORES
ROWS_PER_WORKER = N_ROWS // NUM_WORKERS
CHUNK_ROWS = 512
NUM_CHUNKS = ROWS_PER_WORKER // CHUNK_ROWS
LANES = 16
BLOCKS = N_FEAT // LANES


def _permute_body(inputs_hbm, perm_hbm, out_hbm, perm_v, buf):
    c = lax.axis_index("c")
    s = lax.axis_index("s")
    wid = s * NUM_CORES + c
    base = wid * ROWS_PER_WORKER

    pltpu.sync_copy(perm_hbm, perm_v)
    idxs = [perm_v[pl.ds(k * LANES, LANES)] for k in range(BLOCKS)]

    chunk_elems = CHUNK_ROWS * N_FEAT

    def chunk_body(g, carry):
        elem0 = (base + g * CHUNK_ROWS) * N_FEAT
        pltpu.sync_copy(inputs_hbm.at[pl.ds(elem0, chunk_elems)], buf)

        def row_body(r, carry2):
            rbase = jnp.full((LANES,), r * N_FEAT, jnp.int32)
            vals = [plsc.load_gather(buf, [idxs[k] + rbase])
                    for k in range(BLOCKS)]
            for k in range(BLOCKS):
                buf[pl.ds(r * N_FEAT + k * LANES, LANES)] = vals[k]
            return carry2

        lax.fori_loop(0, CHUNK_ROWS, row_body, 0)
        pltpu.sync_copy(buf, out_hbm.at[pl.ds(elem0, chunk_elems)])
        return carry

    lax.fori_loop(0, NUM_CHUNKS, chunk_body, 0)


@jax.jit
def _permute(inputs_flat, permutation):
    mesh = plsc.VectorSubcoreMesh(core_axis_name="c", subcore_axis_name="s")
    fn = functools.partial(
        pl.kernel,
        out_type=jax.ShapeDtypeStruct((N_ROWS * N_FEAT,), jnp.float32),
        mesh=mesh,
        scratch_types=[
            pltpu.VMEM((N_FEAT,), jnp.int32),
            pltpu.VMEM((CHUNK_ROWS * N_FEAT,), jnp.float32),
        ],
        compiler_params=pltpu.CompilerParams(needs_layout_passes=False),
    )(_permute_body)
    return fn(inputs_flat, permutation)


def kernel(inputs, permutation):
    out_flat = _permute(inputs.reshape(-1), permutation)
    outputs = out_flat.reshape(N_ROWS, N_FEAT)
    logabsdet = jnp.zeros((inputs.shape[0],), dtype=inputs.dtype)
    return (outputs, logabsdet)


# speedup vs baseline: 3.4306x; 1.3341x over previous
import functools

import jax
import jax.numpy as jnp
from jax import lax
from jax.experimental import pallas as pl
from jax.experimental.pallas import tpu as pltpu
from jax.experimental.pallas import tpu_sc as plsc

N_ROWS = 131072
N_FEAT = 128
NUM_CORES = 2
NUM_SUBCORES = 16
NUM_WORKERS = NUM_CORES * NUM_SUBCORES
ROWS_PER_WORKER = N_ROWS // NUM_WORKERS
CHUNK_ROWS = 256
NUM_CHUNKS = ROWS_PER_WORKER // CHUNK_ROWS
LANES = 16
BLOCKS = N_FEAT // LANES
CHUNK_ELEMS = CHUNK_ROWS * N_FEAT


def _permute_body(inputs_hbm, perm_hbm, out_hbm, perm_v, buf0, buf1,
                  sem_in0, sem_in1, sem_out0, sem_out1):
    c = lax.axis_index("c")
    s = lax.axis_index("s")
    wid = s * NUM_CORES + c
    base = wid * ROWS_PER_WORKER

    bufs = (buf0, buf1)
    sems_in = (sem_in0, sem_in1)
    sems_out = (sem_out0, sem_out1)

    pltpu.sync_copy(perm_hbm, perm_v)
    idxs = [perm_v[pl.ds(k * LANES, LANES)] for k in range(BLOCKS)]

    def copy_in(g):
        elem0 = (base + g * CHUNK_ROWS) * N_FEAT
        return pltpu.make_async_copy(
            inputs_hbm.at[pl.ds(elem0, CHUNK_ELEMS)], bufs[g % 2],
            sems_in[g % 2])

    def copy_out(g):
        elem0 = (base + g * CHUNK_ROWS) * N_FEAT
        return pltpu.make_async_copy(
            bufs[g % 2], out_hbm.at[pl.ds(elem0, CHUNK_ELEMS)],
            sems_out[g % 2])

    def compute(b):
        buf = bufs[b]

        def row_body(r, carry2):
            rbase = jnp.full((LANES,), r * N_FEAT, jnp.int32)
            vals = [plsc.load_gather(buf, [idxs[k] + rbase])
                    for k in range(BLOCKS)]
            for k in range(BLOCKS):
                buf[pl.ds(r * N_FEAT + k * LANES, LANES)] = vals[k]
            return carry2

        lax.fori_loop(0, CHUNK_ROWS, row_body, 0)

    copy_in(0).start()
    for g in range(NUM_CHUNKS):
        copy_in(g).wait()
        if g + 1 < NUM_CHUNKS:
            if g >= 1:
                copy_out(g - 1).wait()
            copy_in(g + 1).start()
        compute(g % 2)
        copy_out(g).start()
    copy_out(NUM_CHUNKS - 2).wait()
    copy_out(NUM_CHUNKS - 1).wait()


@jax.jit
def _permute(inputs_flat, permutation):
    mesh = plsc.VectorSubcoreMesh(core_axis_name="c", subcore_axis_name="s")
    fn = functools.partial(
        pl.kernel,
        out_type=jax.ShapeDtypeStruct((N_ROWS * N_FEAT,), jnp.float32),
        mesh=mesh,
        scratch_types=[
            pltpu.VMEM((N_FEAT,), jnp.int32),
            pltpu.VMEM((CHUNK_ELEMS,), jnp.float32),
            pltpu.VMEM((CHUNK_ELEMS,), jnp.float32),
            pltpu.SemaphoreType.DMA,
            pltpu.SemaphoreType.DMA,
            pltpu.SemaphoreType.DMA,
            pltpu.SemaphoreType.DMA,
        ],
        compiler_params=pltpu.CompilerParams(needs_layout_passes=False),
    )(_permute_body)
    return fn(inputs_flat, permutation)


def kernel(inputs, permutation):
    out_flat = _permute(inputs.reshape(-1), permutation)
    outputs = out_flat.reshape(N_ROWS, N_FEAT)
    logabsdet = jnp.zeros((inputs.shape[0],), dtype=inputs.dtype)
    return (outputs, logabsdet)
